# baseline (speedup 1.0000x reference)
import functools

import jax
import jax.numpy as jnp
from jax import lax
from jax.experimental import pallas as pl
from jax.experimental.pallas import tpu as pltpu
from jax.experimental.pallas import tpu_sc as plsc

_NC = 2
_NS = 16
_NW = _NC * _NS


def _gather_sc(y, idx_map):
    batch = y.shape[0]
    vocab = idx_map.shape[0]
    bpw = batch // _NW

    mesh = plsc.VectorSubcoreMesh(core_axis_name="c", subcore_axis_name="s")

    nch = 2
    ch = bpw // nch

    @functools.partial(
        pl.kernel,
        out_type=jax.ShapeDtypeStruct((batch,), jnp.int32),
        mesh=mesh,
        scratch_types=[
            pltpu.VMEM((bpw,), jnp.int32),
            pltpu.VMEM((bpw,), jnp.int32),
            pltpu.VMEM_SHARED((vocab,), jnp.int32),
            pltpu.SemaphoreType.DMA,
            pltpu.SemaphoreType.DMA,
            pltpu.SemaphoreType.DMA,
        ]
        + [pltpu.SemaphoreType.DMA] * nch,
    )
    def k(y_hbm, table_hbm, out_hbm, idx_v, vals_v, tab_s,
          sem_i, sem_t, sem_o, *sem_g):
        sid = lax.axis_index("s")
        wid = sid * _NC + lax.axis_index("c")
        base = wid * bpw
        pltpu.async_copy(y_hbm.at[pl.ds(base, bpw)], idx_v, sem_i)

        @pl.when(sid == 0)
        def _():
            pltpu.async_copy(table_hbm, tab_s, sem_t).wait()

        pltpu.make_async_copy(y_hbm.at[pl.ds(base, bpw)], idx_v, sem_i).wait()
        plsc.subcore_barrier()
        for c in range(nch):
            pltpu.async_copy(
                tab_s.at[idx_v.at[pl.ds(c * ch, ch)]],
                vals_v.at[pl.ds(c * ch, ch)],
                sem_g[c],
            )
        for c in range(nch):
            pltpu.make_async_copy(
                tab_s.at[idx_v.at[pl.ds(c * ch, ch)]],
                vals_v.at[pl.ds(c * ch, ch)],
                sem_g[c],
            ).wait()
            pltpu.async_copy(
                vals_v.at[pl.ds(c * ch, ch)],
                out_hbm.at[pl.ds(base + c * ch, ch)],
                sem_o,
            )
        for c in range(nch):
            pltpu.make_async_copy(
                vals_v.at[pl.ds(c * ch, ch)],
                out_hbm.at[pl.ds(base + c * ch, ch)],
                sem_o,
            ).wait()

    return k(y, idx_map)


def kernel(y, idx_map):
    return _gather_sc(y, idx_map)


# speedup vs baseline: 1.0012x; 1.0012x over previous
import functools

import jax
import jax.numpy as jnp
from jax import lax
from jax.experimental import pallas as pl
from jax.experimental.pallas import tpu as pltpu
from jax.experimental.pallas import tpu_sc as plsc

_NC = 2
_NS = 16
_NW = _NC * _NS


def _gather_sc(y, idx_map):
    batch = y.shape[0]
    vocab = idx_map.shape[0]
    bpw = batch // _NW

    mesh = plsc.VectorSubcoreMesh(core_axis_name="c", subcore_axis_name="s")

    nch = 4
    ch = bpw // nch

    @functools.partial(
        pl.kernel,
        out_type=jax.ShapeDtypeStruct((batch,), jnp.int32),
        mesh=mesh,
        scratch_types=[
            pltpu.VMEM((bpw,), jnp.int32),
            pltpu.VMEM((bpw,), jnp.int32),
            pltpu.VMEM_SHARED((vocab,), jnp.int32),
            pltpu.SemaphoreType.DMA,
            pltpu.SemaphoreType.DMA,
            pltpu.SemaphoreType.DMA,
        ]
        + [pltpu.SemaphoreType.DMA] * nch,
    )
    def k(y_hbm, table_hbm, out_hbm, idx_v, vals_v, tab_s,
          sem_i, sem_t, sem_o, *sem_g):
        sid = lax.axis_index("s")
        wid = sid * _NC + lax.axis_index("c")
        base = wid * bpw
        pltpu.async_copy(y_hbm.at[pl.ds(base, bpw)], idx_v, sem_i)

        @pl.when(sid == 0)
        def _():
            pltpu.async_copy(table_hbm, tab_s, sem_t).wait()

        pltpu.make_async_copy(y_hbm.at[pl.ds(base, bpw)], idx_v, sem_i).wait()
        plsc.subcore_barrier()
        for c in range(nch):
            pltpu.async_copy(
                tab_s.at[idx_v.at[pl.ds(c * ch, ch)]],
                vals_v.at[pl.ds(c * ch, ch)],
                sem_g[c],
            )
        for c in range(nch):
            pltpu.make_async_copy(
                tab_s.at[idx_v.at[pl.ds(c * ch, ch)]],
                vals_v.at[pl.ds(c * ch, ch)],
                sem_g[c],
            ).wait()
            pltpu.async_copy(
                vals_v.at[pl.ds(c * ch, ch)],
                out_hbm.at[pl.ds(base + c * ch, ch)],
                sem_o,
            )
        for c in range(nch):
            pltpu.make_async_copy(
                vals_v.at[pl.ds(c * ch, ch)],
                out_hbm.at[pl.ds(base + c * ch, ch)],
                sem_o,
            ).wait()

    return k(y, idx_map)


def kernel(y, idx_map):
    return _gather_sc(y, idx_map)
